# baseline (speedup 1.0000x reference)
import functools

import jax
import jax.numpy as jnp
from jax import lax
from jax.experimental import pallas as pl
from jax.experimental.pallas import tpu as pltpu
from jax.experimental.pallas import tpu_sc as plsc

EMB = 64
BATCH = 4096
OUT = 128
VOCAB = 100000
NC, NS = 2, 16
NW = NC * NS
BPW = BATCH // NW
L = 16

_BV = 4096
_NB = 13
HALF = _NB * _BV


def _make_proj_body(t0):
    def _proj_body(tal, tah, tbl, tbh, w_ref, pa, pb):
        dn = (((0,), (0,)), ((), ()))
        ins = ((tal, tah, pa, t0), (tbl, tbh, pb, t0 + 1))
        for tlo, thi, pout, t in ins:
            wt = w_ref[pl.ds(t * EMB, EMB), :].astype(jnp.bfloat16)
            lo = lax.dot_general(tlo[...].astype(jnp.bfloat16), wt, dn,
                                 preferred_element_type=jnp.float32)
            hi = lax.dot_general(thi[...].astype(jnp.bfloat16), wt, dn,
                                 preferred_element_type=jnp.float32)
            lo16 = lax.bitcast_convert_type(lo.astype(jnp.bfloat16),
                                            jnp.uint16)
            hi16 = lax.bitcast_convert_type(hi.astype(jnp.bfloat16),
                                            jnp.uint16)
            word = (lax.shift_left(hi16.astype(jnp.uint32), jnp.uint32(16))
                    | lo16.astype(jnp.uint32))
            pout[...] = lax.bitcast_convert_type(word, jnp.int32)
    return _proj_body


def _tc_project2(Ta, Tb, W, t0):
    lo_spec = pl.BlockSpec((EMB, _BV), lambda i: (0, i))
    hi_spec = pl.BlockSpec(
        (EMB, _BV), lambda i: (0, jnp.minimum(i + _NB, (VOCAB - 1) // _BV)))
    p_spec = pl.BlockSpec((_BV, OUT), lambda i: (i, 0))
    return pl.pallas_call(
        _make_proj_body(t0),
        grid=(_NB,),
        in_specs=[lo_spec, hi_spec, lo_spec, hi_spec,
                  pl.BlockSpec((4 * EMB, OUT), lambda i: (0, 0))],
        out_specs=[p_spec] * 2,
        out_shape=[jax.ShapeDtypeStruct((HALF, OUT), jnp.int32)] * 2,
    )(Ta, Ta, Tb, Tb, W)


_sc_mesh = plsc.VectorSubcoreMesh(core_axis_name="c", subcore_axis_name="s")


@functools.partial(
    pl.kernel,
    out_type=tuple(
        jax.ShapeDtypeStruct((BATCH, OUT), jnp.int32) for _ in range(2)
    ),
    mesh=_sc_mesh,
    scratch_types=[
        pltpu.VMEM((BPW,), jnp.int32),
        pltpu.VMEM((BPW,), jnp.int32),
        pltpu.VMEM((BPW, OUT), jnp.int32),
        pltpu.SemaphoreType.DMA,
    ],
)
def _sc_gather2(xa, xb, Pa, Pb, ga, gb, idx_v, j_v, rows_v, sem):
    wid = lax.axis_index("s") * NC + lax.axis_index("c")
    base = wid * BPW
    for x, P, g in ((xa, Pa, ga), (xb, Pb, gb)):
        pltpu.sync_copy(x.at[pl.ds(base, BPW)], idx_v)
        for i in range(BPW // L):
            xv = idx_v[pl.ds(i * L, L)]
            j_v[pl.ds(i * L, L)] = jnp.where(xv >= HALF, xv - HALF, xv)
        pltpu.async_copy(P.at[j_v], rows_v, sem).wait()
        pltpu.sync_copy(rows_v, g.at[pl.ds(base, BPW)])


_BM = 1024


def _sum_body(ga, gb, gc, gd, xa, xb, xc, xd, b_ref, out_ref):
    acc = b_ref[...] + jnp.zeros((_BM, OUT), jnp.float32)
    for g, x in ((ga, xa), (gb, xb), (gc, xc), (gd, xd)):
        is_lo = x[...] < HALF
        u = lax.bitcast_convert_type(g[...], jnp.uint32)
        half = jnp.where(is_lo, u & jnp.uint32(0xFFFF),
                         lax.shift_right_logical(u, jnp.uint32(16)))
        val = lax.bitcast_convert_type(
            half.astype(jnp.uint16), jnp.bfloat16).astype(jnp.float32)
        acc = acc + val
    out_ref[...] = acc


def _tc_sum(gs, xs2d, b2d):
    g_spec = pl.BlockSpec((_BM, OUT), lambda i: (i, 0))
    x_spec = pl.BlockSpec((_BM, 1), lambda i: (i, 0))
    return pl.pallas_call(
        _sum_body,
        grid=(BATCH // _BM,),
        in_specs=[g_spec] * 4 + [x_spec] * 4
        + [pl.BlockSpec((1, OUT), lambda i: (0, 0))],
        out_specs=g_spec,
        out_shape=jax.ShapeDtypeStruct((BATCH, OUT), jnp.float32),
    )(*gs, *xs2d, b2d)


def kernel(x_a, x_b, x_c, x_d, T_a, T_b, T_c, T_d, W, b):
    xs = [x.astype(jnp.int32) for x in (x_a, x_b, x_c, x_d)]
    Pa, Pb = _tc_project2(T_a.T, T_b.T, W, 0)
    ga, gb = _sc_gather2(xs[0], xs[1], Pa, Pb)
    Pc, Pd = _tc_project2(T_c.T, T_d.T, W, 2)
    gc_, gd = _sc_gather2(xs[2], xs[3], Pc, Pd)
    xs2d = [x.reshape(BATCH, 1) for x in xs]
    return _tc_sum([ga, gb, gc_, gd], xs2d, b.reshape(1, OUT))


# speedup vs baseline: 1.0523x; 1.0523x over previous
import functools

import jax
import jax.numpy as jnp
from jax import lax
from jax.experimental import pallas as pl
from jax.experimental.pallas import tpu as pltpu
from jax.experimental.pallas import tpu_sc as plsc

EMB = 64
BATCH = 4096
OUT = 128
VOCAB = 100000
NC, NS = 2, 16
NW = NC * NS
BPW = BATCH // NW
L = 16

_BV = 4096
_NB = 13
HALF = _NB * _BV


def _proj_body(tal, tah, tbl, tbh, tcl, tch, tdl, tdh, w_ref,
               pa, pb, pc, pd):
    dn = (((0,), (0,)), ((), ()))
    ins = ((tal, tah, pa), (tbl, tbh, pb), (tcl, tch, pc), (tdl, tdh, pd))
    for t, (tlo, thi, pout) in enumerate(ins):
        wt = w_ref[pl.ds(t * EMB, EMB), :].astype(jnp.bfloat16)
        lo = lax.dot_general(tlo[...].astype(jnp.bfloat16), wt, dn,
                             preferred_element_type=jnp.float32)
        hi = lax.dot_general(thi[...].astype(jnp.bfloat16), wt, dn,
                             preferred_element_type=jnp.float32)
        lo16 = lax.bitcast_convert_type(lo.astype(jnp.bfloat16), jnp.uint16)
        hi16 = lax.bitcast_convert_type(hi.astype(jnp.bfloat16), jnp.uint16)
        word = (lax.shift_left(hi16.astype(jnp.uint32), jnp.uint32(16))
                | lo16.astype(jnp.uint32))
        pout[...] = lax.bitcast_convert_type(word, jnp.int32)


def _tc_project(Tts, W):
    lo_spec = pl.BlockSpec((EMB, _BV), lambda i: (0, i))
    hi_spec = pl.BlockSpec((EMB, _BV),
                           lambda i: (0, jnp.minimum(i + _NB, (VOCAB - 1) // _BV)))
    p_spec = pl.BlockSpec((_BV, OUT), lambda i: (i, 0))
    in_specs = []
    for _ in range(4):
        in_specs += [lo_spec, hi_spec]
    return pl.pallas_call(
        _proj_body,
        grid=(_NB,),
        in_specs=in_specs + [pl.BlockSpec((4 * EMB, OUT), lambda i: (0, 0))],
        out_specs=[p_spec] * 4,
        out_shape=[jax.ShapeDtypeStruct((HALF, OUT), jnp.int32)] * 4,
    )(*[T for T in Tts for _ in range(2)], W)


_sc_mesh = plsc.VectorSubcoreMesh(core_axis_name="c", subcore_axis_name="s")


@functools.partial(
    pl.kernel,
    out_type=tuple(
        jax.ShapeDtypeStruct((BATCH, OUT), jnp.int32) for _ in range(4)
    ),
    mesh=_sc_mesh,
    scratch_types=[
        pltpu.VMEM((BPW,), jnp.int32),
        [pltpu.VMEM((BPW,), jnp.int32) for _ in range(4)],
        [pltpu.VMEM((BPW, OUT), jnp.int32) for _ in range(4)],
        pltpu.SemaphoreType.DMA,
    ],
)
def _sc_gather4(xa, xb, xc, xd, Pa, Pb, Pc, Pd, ga, gb, gc, gd,
                idx_v, j_vs, rows_vs, sem):
    wid = lax.axis_index("s") * NC + lax.axis_index("c")
    base = wid * BPW
    tabs = ((xa, Pa, ga), (xb, Pb, gb), (xc, Pc, gc), (xd, Pd, gd))
    for t, (x, P, g) in enumerate(tabs):
        pltpu.sync_copy(x.at[pl.ds(base, BPW)], idx_v)
        for i in range(BPW // L):
            xv = idx_v[pl.ds(i * L, L)]
            j_vs[t][pl.ds(i * L, L)] = jnp.where(xv >= HALF, xv - HALF, xv)
        pltpu.async_copy(P.at[j_vs[t]], rows_vs[t], sem)
    for t, (x, P, g) in enumerate(tabs):
        pltpu.make_async_copy(P.at[j_vs[t]], rows_vs[t], sem).wait()
        pltpu.sync_copy(rows_vs[t], g.at[pl.ds(base, BPW)])


_BM = 1024


def _sum_body(ga, gb, gc, gd, xa, xb, xc, xd, b_ref, out_ref):
    acc = b_ref[...] + jnp.zeros((_BM, OUT), jnp.float32)
    for g, x in ((ga, xa), (gb, xb), (gc, xc), (gd, xd)):
        is_lo = x[...] < HALF
        u = lax.bitcast_convert_type(g[...], jnp.uint32)
        half = jnp.where(is_lo, u & jnp.uint32(0xFFFF),
                         lax.shift_right_logical(u, jnp.uint32(16)))
        val = lax.bitcast_convert_type(
            half.astype(jnp.uint16), jnp.bfloat16).astype(jnp.float32)
        acc = acc + val
    out_ref[...] = acc


def _tc_sum(gs, xs2d, b2d):
    g_spec = pl.BlockSpec((_BM, OUT), lambda i: (i, 0))
    x_spec = pl.BlockSpec((_BM, 1), lambda i: (i, 0))
    return pl.pallas_call(
        _sum_body,
        grid=(BATCH // _BM,),
        in_specs=[g_spec] * 4 + [x_spec] * 4
        + [pl.BlockSpec((1, OUT), lambda i: (0, 0))],
        out_specs=g_spec,
        out_shape=jax.ShapeDtypeStruct((BATCH, OUT), jnp.float32),
    )(*gs, *xs2d, b2d)


def kernel(x_a, x_b, x_c, x_d, T_a, T_b, T_c, T_d, W, b):
    xs = [x.astype(jnp.int32) for x in (x_a, x_b, x_c, x_d)]
    Ps = _tc_project([T.T for T in (T_a, T_b, T_c, T_d)], W)
    gs = _sc_gather4(*xs, *Ps)
    xs2d = [x.reshape(BATCH, 1) for x in xs]
    return _tc_sum(gs, xs2d, b.reshape(1, OUT))
